# baseline (speedup 1.0000x reference)
import functools

import jax
import jax.numpy as jnp
from jax import lax
from jax.experimental import pallas as pl
from jax.experimental.pallas import tpu as pltpu
from jax.experimental.pallas import tpu_sc as plsc

_BB = 8
_NB_SC = 8


def _tc_body(x_ref, der_ref, out_ref):
    for i in range(_BB):
        xb = jnp.broadcast_to(x_ref[0, i], (8, x_ref.shape[-1]))
        y = lax.dot_general(
            xb, der_ref[i],
            dimension_numbers=(((1,), (1,)), ((), ())),
            preferred_element_type=jnp.float32,
        )
        y0 = y[0]
        out_ref[0, i] = y0 * y0


def _tc_call(x3, der_t, nb_tc):
    D = x3.shape[-1]
    RT = der_t.shape[1]
    return pl.pallas_call(
        _tc_body,
        grid=(nb_tc // _BB,),
        in_specs=[
            pl.BlockSpec((1, _BB, D), lambda b: (b, 0, 0)),
            pl.BlockSpec((_BB, RT, D), lambda b: (b, 0, 0)),
        ],
        out_specs=pl.BlockSpec((1, _BB, RT), lambda b: (b, 0, 0)),
        out_shape=jax.ShapeDtypeStruct((nb_tc // _BB, _BB, RT), jnp.float32),
    )(x3, der_t[:nb_tc])


def _sc_call(x, der_f, nb, RT, D):
    B = der_f.shape[0]
    spb = 32 // nb
    rows = RT // spb
    b0 = B - nb
    mesh = plsc.VectorSubcoreMesh(core_axis_name="c", subcore_axis_name="s")

    @functools.partial(
        pl.kernel,
        out_type=jax.ShapeDtypeStruct((nb, RT), jnp.float32),
        mesh=mesh,
        scratch_types=[
            pltpu.VMEM((rows * D,), jnp.float32),
            pltpu.VMEM((D,), jnp.float32),
            pltpu.VMEM((rows,), jnp.float32),
        ],
    )
    def sck(x_hbm, der_hbm, out_hbm, buf, xrow, orow):
        c = lax.axis_index("c")
        s = lax.axis_index("s")
        w = s * 2 + c
        bl = w // spb
        b = b0 + bl
        r0 = (w % spb) * rows
        pltpu.sync_copy(x_hbm.at[b], xrow)
        pltpu.sync_copy(der_hbm.at[b, pl.ds(r0 * D, rows * D)], buf)
        iotav = lax.iota(jnp.int32, 16)
        xv = [xrow[pl.ds(v * 16, 16)] for v in range(D // 16)]

        def grp(g, tok):
            gbase = g * (16 * D)
            acc = jnp.zeros((16,), jnp.float32)
            for l in range(16):
                rbase = gbase + l * D
                p = buf[pl.ds(rbase, 16)] * xv[0]
                for v in range(1, D // 16):
                    p = p + buf[pl.ds(rbase + v * 16, 16)] * xv[v]
                for k in (1, 2, 4, 8):
                    p = p + jnp.take(p, jnp.bitwise_xor(iotav, k))
                acc = jnp.where(iotav == l, p, acc)
            orow[pl.ds(g * 16, 16)] = acc * acc
            return tok

        lax.fori_loop(0, rows // 16, grp, 0)
        pltpu.sync_copy(orow, out_hbm.at[bl, pl.ds(r0, rows)])

    return sck(x, der_f)


def kernel(x, der_desc_wrt_pos):
    B, A, D, T = der_desc_wrt_pos.shape
    RT = T * A
    der_t = der_desc_wrt_pos.transpose(0, 3, 1, 2).reshape(B, RT, D)
    nb_tc = B - _NB_SC
    x3 = x[:nb_tc].reshape(nb_tc // _BB, _BB, D)
    out_tc = _tc_call(x3, der_t, nb_tc).reshape(nb_tc, RT)
    out_sc = _sc_call(x, der_t.reshape(B, RT * D), _NB_SC, RT, D)
    out = jnp.concatenate([out_tc, out_sc], axis=0)
    return out.reshape(B, T, A).transpose(0, 2, 1).reshape(B, A * T)


# speedup vs baseline: 2.1125x; 2.1125x over previous
import functools

import jax
import jax.numpy as jnp
from jax import lax
from jax.experimental import pallas as pl
from jax.experimental.pallas import tpu as pltpu
from jax.experimental.pallas import tpu_sc as plsc

_BB = 8
_NB_SC = 8


def _tc_body(x_ref, der_ref, out_ref):
    for i in range(_BB):
        xb = jnp.broadcast_to(x_ref[0, i], (8, x_ref.shape[-1]))
        y = lax.dot_general(
            xb, der_ref[i],
            dimension_numbers=(((1,), (1,)), ((), ())),
            preferred_element_type=jnp.float32,
        )
        y0 = y[0]
        out_ref[0, i] = y0 * y0


def _tc_call(x3, der_t, nb_tc):
    D = x3.shape[-1]
    RT = der_t.shape[1]
    return pl.pallas_call(
        _tc_body,
        grid=(nb_tc // _BB,),
        in_specs=[
            pl.BlockSpec((1, _BB, D), lambda b: (b, 0, 0)),
            pl.BlockSpec((_BB, RT, D), lambda b: (b, 0, 0)),
        ],
        out_specs=pl.BlockSpec((1, _BB, RT), lambda b: (b, 0, 0)),
        out_shape=jax.ShapeDtypeStruct((nb_tc // _BB, _BB, RT), jnp.float32),
    )(x3, der_t)


def _sc_call(x, der_t, nb):
    B, RT, D = der_t.shape
    spb = 32 // nb
    rows = RT // spb
    b0 = B - nb
    mesh = plsc.VectorSubcoreMesh(core_axis_name="c", subcore_axis_name="s")

    @functools.partial(
        pl.kernel,
        out_type=jax.ShapeDtypeStruct((nb, RT), jnp.float32),
        mesh=mesh,
        scratch_types=[
            pltpu.VMEM((rows, D), jnp.float32),
            pltpu.VMEM((D,), jnp.float32),
            pltpu.VMEM((rows,), jnp.float32),
        ],
    )
    def sck(x_hbm, der_hbm, out_hbm, buf, xrow, orow):
        c = lax.axis_index("c")
        s = lax.axis_index("s")
        w = s * 2 + c
        bl = w // spb
        b = b0 + bl
        r0 = (w % spb) * rows
        pltpu.sync_copy(x_hbm.at[b], xrow)
        pltpu.sync_copy(der_hbm.at[b, pl.ds(r0, rows)], buf)
        iotav = lax.iota(jnp.int32, 16)
        xv = [xrow[pl.ds(v * 16, 16)] for v in range(D // 16)]

        def grp(g, tok):
            acc = jnp.zeros((16,), jnp.float32)
            for l in range(16):
                r = g * 16 + l
                p = buf[r, pl.ds(0, 16)] * xv[0]
                for v in range(1, D // 16):
                    p = p + buf[r, pl.ds(v * 16, 16)] * xv[v]
                for k in (1, 2, 4, 8):
                    p = p + jnp.take(p, jnp.bitwise_xor(iotav, k))
                acc = jnp.where(iotav == l, p, acc)
            orow[pl.ds(g * 16, 16)] = acc * acc
            return tok

        lax.fori_loop(0, rows // 16, grp, 0)
        pltpu.sync_copy(orow, out_hbm.at[bl, pl.ds(r0, rows)])

    return sck(x, der_t)


def kernel(x, der_desc_wrt_pos):
    B, A, D, T = der_desc_wrt_pos.shape
    RT = T * A
    der_t = der_desc_wrt_pos.transpose(0, 3, 1, 2).reshape(B, RT, D)
    nb_tc = B - _NB_SC
    x3 = x[:nb_tc].reshape(nb_tc // _BB, _BB, D)
    out_tc = _tc_call(x3, der_t, nb_tc).reshape(nb_tc, RT)
    out_sc = _sc_call(x, der_t, _NB_SC)
    out = jnp.concatenate([out_tc, out_sc], axis=0)
    return out.reshape(B, T, A).transpose(0, 2, 1).reshape(B, A * T)


# speedup vs baseline: 4.5108x; 2.1353x over previous
import jax
import jax.numpy as jnp
from jax import lax
from jax.experimental import pallas as pl

_BB = 8
_NC = 2


def _body(x_ref, der_ref, out_ref):
    for i in range(_BB):
        xb = jnp.broadcast_to(x_ref[0, i], (8, x_ref.shape[-1]))
        y = lax.dot_general(
            xb, der_ref[i],
            dimension_numbers=(((1,), (1,)), ((), ())),
            preferred_element_type=jnp.float32,
        )
        y0 = y[0]
        out_ref[0, i] = y0 * y0


def kernel(x, der_desc_wrt_pos):
    B, A, D, T = der_desc_wrt_pos.shape
    RT = T * A
    RC = RT // _NC
    der_t = der_desc_wrt_pos.transpose(0, 3, 1, 2).reshape(B, RT, D)
    x3 = x.reshape(B // _BB, _BB, D)
    out = pl.pallas_call(
        _body,
        grid=(B // _BB, _NC),
        in_specs=[
            pl.BlockSpec((1, _BB, D), lambda b, c: (b, 0, 0)),
            pl.BlockSpec((_BB, RC, D), lambda b, c: (b, c, 0)),
        ],
        out_specs=pl.BlockSpec((1, _BB, RC), lambda b, c: (b, 0, c)),
        out_shape=jax.ShapeDtypeStruct((B // _BB, _BB, RT), jnp.float32),
    )(x3, der_t)
    return out.reshape(B, T, A).transpose(0, 2, 1).reshape(B, A * T)


# speedup vs baseline: 5.0612x; 1.1220x over previous
import jax
import jax.numpy as jnp
from jax import lax
from jax.experimental import pallas as pl

_BB = 8
_NC = 1


def _body(x_ref, der_ref, out_ref):
    for i in range(_BB):
        xb = jnp.broadcast_to(x_ref[0, i], (8, x_ref.shape[-1]))
        y = lax.dot_general(
            xb, der_ref[i],
            dimension_numbers=(((1,), (1,)), ((), ())),
            preferred_element_type=jnp.float32,
        )
        y0 = y[0]
        out_ref[0, i] = y0 * y0


def kernel(x, der_desc_wrt_pos):
    B, A, D, T = der_desc_wrt_pos.shape
    RT = T * A
    RC = RT // _NC
    der_t = der_desc_wrt_pos.transpose(0, 3, 1, 2).reshape(B, RT, D)
    x3 = x.reshape(B // _BB, _BB, D)
    out = pl.pallas_call(
        _body,
        grid=(B // _BB, _NC),
        in_specs=[
            pl.BlockSpec((1, _BB, D), lambda b, c: (b, 0, 0)),
            pl.BlockSpec((_BB, RC, D), lambda b, c: (b, c, 0)),
        ],
        out_specs=pl.BlockSpec((1, _BB, RC), lambda b, c: (b, 0, c)),
        out_shape=jax.ShapeDtypeStruct((B // _BB, _BB, RT), jnp.float32),
    )(x3, der_t)
    return out.reshape(B, T, A).transpose(0, 2, 1).reshape(B, A * T)
